# Initial kernel scaffold; baseline (speedup 1.0000x reference)
import jax
import jax.numpy as jnp
from jax.experimental import pallas as pl


def kernel(y_true0, y_pred0, y_true1, y_pred1, log_vars, eps0, eps1):
    raise NotImplementedError("write your pallas kernel here")


# speedup vs baseline: 3.2482x; 3.2482x over previous
import jax
import jax.numpy as jnp
from jax.experimental import pallas as pl
from jax.experimental.pallas import tpu as pltpu

_P = 32


def _loss_kernel(eps0_ref, eps1_ref, aux_ref, out_ref):
    def task(eps_ref, base):
        a0 = aux_ref[base + 0:base + 1, :]
        a1 = aux_ref[base + 1:base + 2, :]
        a2 = aux_ref[base + 2:base + 3, :]
        sc = aux_ref[base + 3:base + 4, :]
        w0 = aux_ref[base + 4:base + 5, :]
        w1 = aux_ref[base + 5:base + 6, :]
        w2 = aux_ref[base + 6:base + 7, :]
        yt = aux_ref[base + 7:base + 8, :]
        x = eps_ref[...]
        d0 = a0 + x[0] * sc
        d1 = a1 + x[1] * sc
        d2 = a2 + x[2] * sc
        e = jnp.exp(d0) + jnp.exp(d1) + jnp.exp(d2)
        lse = jnp.log(jnp.maximum(e, 1e-30))
        contrib = yt * lse - (w0 * d0 + w1 * d1 + w2 * d2)
        return jnp.sum(contrib, axis=0, keepdims=True)

    r0 = task(eps0_ref, 0)
    r1 = task(eps1_ref, 8)
    out_ref[0] = jnp.concatenate([r0, r1], axis=0)


def _aux_rows(y_true, y_pred):
    lg = y_pred[:, :3].T
    sc = jnp.exp(0.5 * y_pred[:, 3])[None, :]
    w = y_true.T
    yt = jnp.sum(y_true, axis=1)[None, :]
    return jnp.concatenate([lg, sc, w, yt], axis=0)


def kernel(y_true0, y_pred0, y_true1, y_pred1, log_vars, eps0, eps1):
    t, n, _ = eps0.shape
    nb = n // _P

    e0 = jnp.transpose(eps0, (2, 0, 1))
    e1 = jnp.transpose(eps1, (2, 0, 1))
    aux = jnp.concatenate(
        [_aux_rows(y_true0, y_pred0), _aux_rows(y_true1, y_pred1)], axis=0)

    out = pl.pallas_call(
        _loss_kernel,
        grid=(_P,),
        in_specs=[
            pl.BlockSpec((3, t, nb), lambda p: (0, 0, p)),
            pl.BlockSpec((3, t, nb), lambda p: (0, 0, p)),
            pl.BlockSpec((16, nb), lambda p: (0, p)),
        ],
        out_specs=pl.BlockSpec((1, 2, nb), lambda p: (p, 0, 0)),
        out_shape=jax.ShapeDtypeStruct((_P, 2, nb), jnp.float32),
        compiler_params=pltpu.CompilerParams(
            dimension_semantics=("parallel",)),
    )(e0, e1, aux)

    inv_tn = 1.0 / (t * n)
    mc0 = jnp.sum(out[:, 0, :]) * inv_tn
    mc1 = jnp.sum(out[:, 1, :]) * inv_tn
    lv0, lv1 = log_vars[0], log_vars[1]
    return jnp.exp(-lv0) * mc0 + lv0 + jnp.exp(-lv1) * mc1 + lv1
